# baseline (speedup 1.0000x reference)
import functools
import jax
import jax.numpy as jnp
from jax import lax
from jax.experimental import pallas as pl
from jax.experimental.pallas import tpu as pltpu
from jax.experimental.pallas import tpu_sc as plsc

N = 10000
E = 160000
ER = 32000
F_IN = 128
F_OUT = 256
HIDDEN = 128
OUT_DIM = 12
PERIODS = 12

ROWS = 400
GRID = N // ROWS


def _dense_body(xt_ref, O_ref, dinv2_ref, C0_ref, M_ref, U_ref, V_ref,
                cvec_ref, probs_ref, W1_ref, b1_ref, W2_ref, b2_ref,
                out_ref, hid_ref):
    f32 = jnp.float32
    Hacc = jnp.zeros((ROWS, F_OUT), f32)
    c1 = cvec_ref[0:1, :]
    cz = cvec_ref[1:2, :]
    cr = cvec_ref[2:3, :]
    ch = cvec_ref[3:4, :]
    for t in range(PERIODS):
        xp = xt_ref[t]
        h = jnp.dot(xp, C0_ref[...], preferred_element_type=f32) + c1
        for r in range(5):
            h = h + jnp.dot(O_ref[t, 1 + r], M_ref[r], preferred_element_type=f32)
        h = jnp.where(h >= 0, h, 0.01 * h)
        g = O_ref[t, 0] + dinv2_ref[...] * xp
        Z = jax.nn.sigmoid(jnp.dot(g, U_ref[0], preferred_element_type=f32)
                           + jnp.dot(h, V_ref[0], preferred_element_type=f32) + cz)
        Rg = jax.nn.sigmoid(jnp.dot(g, U_ref[1], preferred_element_type=f32)
                            + jnp.dot(h, V_ref[1], preferred_element_type=f32) + cr)
        Ht = jnp.tanh(jnp.dot(g, U_ref[2], preferred_element_type=f32)
                      + jnp.dot(h * Rg, V_ref[2], preferred_element_type=f32) + ch)
        Hn = Z * h + (1.0 - Z) * Ht
        Hacc = Hacc + probs_ref[0, t] * Hn
    hid_ref[...] = Hacc
    hh = jnp.maximum(Hacc, 0.0)
    hh = jnp.maximum(jnp.dot(hh, W1_ref[...], preferred_element_type=f32)
                     + b1_ref[...], 0.0)
    out_ref[...] = jnp.dot(hh, W2_ref[...], preferred_element_type=f32) + b2_ref[...]


def _dense_stage(xt, O, dinv2, C0, M, U, V, cvec, probs, W1, b1, W2, b2,
                 interpret=False):
    full = lambda *shape: pl.BlockSpec(shape, lambda i: (0,) * len(shape))
    return pl.pallas_call(
        _dense_body,
        grid=(GRID,),
        in_specs=[
            pl.BlockSpec((PERIODS, ROWS, F_IN), lambda i: (0, i, 0)),
            pl.BlockSpec((PERIODS, 6, ROWS, F_IN), lambda i: (0, 0, i, 0)),
            pl.BlockSpec((ROWS, 1), lambda i: (i, 0)),
            full(F_IN, F_OUT),
            full(5, F_IN, F_OUT),
            full(3, F_IN, F_OUT),
            full(3, F_OUT, F_OUT),
            full(4, F_OUT),
            full(1, PERIODS),
            full(F_OUT, HIDDEN),
            full(1, HIDDEN),
            full(HIDDEN, OUT_DIM),
            full(1, OUT_DIM),
        ],
        out_specs=[
            pl.BlockSpec((ROWS, OUT_DIM), lambda i: (i, 0)),
            pl.BlockSpec((ROWS, F_OUT), lambda i: (i, 0)),
        ],
        out_shape=[
            jax.ShapeDtypeStruct((N, OUT_DIM), jnp.float32),
            jax.ShapeDtypeStruct((N, F_OUT), jnp.float32),
        ],
        interpret=interpret,
    )(xt, O, dinv2, C0, M, U, V, cvec, probs, W1, b1, W2, b2)


CH = 128
NS = 16
TROWS = 624
ZR = 104
GCH = 1280
RCH = 256
TOTCH = GCH + 5 * RCH


def _spmm_body(half, xflat, spk, dpk, wpk, zeros_hbm, o_hbm,
               s0, s1, d0, d1, w0, w1, r0b, r1b, zbuf, acc,
               semE0, semE1, semG0, semG1):
    sid = lax.axis_index("s")
    row0 = pl.multiple_of(sid * TROWS, 8)
    pltpu.sync_copy(zeros_hbm, zbuf)

    def scale(rb, wb):
        def grp(g, _):
            wv = wb[pl.ds(16 * g, 16)]
            for l in range(16):
                we = wv[l]
                e = 16 * g + l
                for j in range(F_IN // 16):
                    rb[e, pl.ds(16 * j, 16)] = rb[e, pl.ds(16 * j, 16)] * we
            return 0
        lax.fori_loop(0, CH // 16, grp, 0)

    def prep(sb, toff):
        for j in range(CH // 16):
            sb[pl.ds(16 * j, 16)] = sb[pl.ds(16 * j, 16)] + toff

    def do_pass(t, out_base, cbase, niter):
        for k in range(6):
            pltpu.sync_copy(zbuf, acc.at[pl.ds(row0 + ZR * k, ZR)])
        @pl.when(sid == NS - 1)
        def _():
            pltpu.sync_copy(zbuf.at[pl.ds(0, 16)], acc.at[pl.ds(row0 + 6 * ZR, 16)])
        plsc.subcore_barrier()

        toff = t * N

        def startE(i, sb, db, wb, sem):
            c = pl.multiple_of((cbase + sid + NS * i) * CH, 8)
            pltpu.async_copy(spk.at[pl.ds(c, CH)], sb, sem)
            pltpu.async_copy(dpk.at[pl.ds(c, CH)], db, sem)
            pltpu.async_copy(wpk.at[pl.ds(c, CH)], wb, sem)

        def waitE(sb, db, wb, sem):
            pltpu.make_async_copy(spk.at[pl.ds(0, CH)], sb, sem).wait()
            pltpu.make_async_copy(dpk.at[pl.ds(0, CH)], db, sem).wait()
            pltpu.make_async_copy(wpk.at[pl.ds(0, CH)], wb, sem).wait()

        startG = lambda sb, rb, sem: pltpu.async_copy(xflat.at[sb], rb, sem)

        startE(0, s0, d0, w0, semE0)
        startE(1, s1, d1, w1, semE1)
        waitE(s0, d0, w0, semE0)
        prep(s0, toff)
        startG(s0, r0b, semG0)

        def lbody(i2, _):
            iA = 2 * i2
            pltpu.make_async_copy(xflat.at[s0], r0b, semG0).wait()
            waitE(s1, d1, w1, semE1)
            prep(s1, toff)
            startG(s1, r1b, semG1)
            scale(r0b, w0)
            pltpu.sync_copy(r0b, acc.at[d0], add=True)
            @pl.when(iA + 2 < niter)
            def _():
                startE(iA + 2, s0, d0, w0, semE0)
            pltpu.make_async_copy(xflat.at[s1], r1b, semG1).wait()
            @pl.when(iA + 2 < niter)
            def _():
                waitE(s0, d0, w0, semE0)
                prep(s0, toff)
                startG(s0, r0b, semG0)
            scale(r1b, w1)
            pltpu.sync_copy(r1b, acc.at[d1], add=True)
            @pl.when(iA + 3 < niter)
            def _():
                startE(iA + 3, s1, d1, w1, semE1)
            return 0
        lax.fori_loop(0, niter // 2, lbody, 0)
        plsc.subcore_barrier()

        ob = pl.multiple_of(out_base + row0, 8)
        @pl.when(sid < NS - 1)
        def _():
            pltpu.sync_copy(acc.at[pl.ds(row0, TROWS)], o_hbm.at[pl.ds(ob, TROWS)])
        @pl.when(sid == NS - 1)
        def _():
            pltpu.sync_copy(acc.at[pl.ds(row0, TROWS + 16)],
                            o_hbm.at[pl.ds(ob, TROWS + 16)])
        plsc.subcore_barrier()

    def per_t(tp, _):
        t = 6 * half + tp
        do_pass(t, (tp * 6) * N, 0, GCH // NS)
        def per_r(r, _):
            do_pass(t, (tp * 6 + 1 + r) * N, GCH + RCH * r, RCH // NS)
            return 0
        lax.fori_loop(0, 5, per_r, 0)
        return 0
    lax.fori_loop(0, 6, per_t, 0)


def _spmm_sc(xflat, spk, dpk, wpk, zeros_hbm, half):
    mesh = plsc.VectorSubcoreMesh(core_axis_name="c", subcore_axis_name="s",
                                  num_cores=1)
    f = pl.kernel(
        functools.partial(_spmm_body, half),
        out_type=jax.ShapeDtypeStruct((PERIODS * 3 * N, F_IN), jnp.float32),
        mesh=mesh,
        scratch_types=[
            pltpu.VMEM((CH,), jnp.int32),
            pltpu.VMEM((CH,), jnp.int32),
            pltpu.VMEM((CH,), jnp.int32),
            pltpu.VMEM((CH,), jnp.int32),
            pltpu.VMEM((CH,), jnp.float32),
            pltpu.VMEM((CH,), jnp.float32),
            pltpu.VMEM((CH, F_IN), jnp.float32),
            pltpu.VMEM((CH, F_IN), jnp.float32),
            pltpu.VMEM((ZR, F_IN), jnp.float32),
            pltpu.VMEM_SHARED((N, F_IN), jnp.float32),
            pltpu.SemaphoreType.DMA,
            pltpu.SemaphoreType.DMA,
            pltpu.SemaphoreType.DMA,
            pltpu.SemaphoreType.DMA,
        ],
    )
    return f(xflat, spk, dpk, wpk, zeros_hbm)


def _pack_edges(src, dst, w, nch):
    npad = nch * CH - src.shape[0]
    src = jnp.concatenate([src, jnp.zeros((npad,), jnp.int32)])
    dst = jnp.concatenate([dst, jnp.zeros((npad,), jnp.int32)])
    w = jnp.concatenate([w, jnp.zeros((npad,), jnp.float32)])
    return src, dst, w


def kernel(x, edge_index, IAedge_index, KSedge_index, KYedge_index,
           OHedge_index, WIedge_index, IAedge_attr, KSedge_attr, KYedge_attr,
           OHedge_attr, WIedge_attr, params):
    p = params
    reg_ei = (IAedge_index, KSedge_index, KYedge_index, OHedge_index, WIedge_index)
    reg_ew = (IAedge_attr, KSedge_attr, KYedge_attr, OHedge_attr, WIedge_attr)

    norms = []
    for ei, ew in zip(reg_ei, reg_ew):
        src, dst = ei[0], ei[1]
        deg = jnp.zeros((N,), jnp.float32).at[src].add(ew)
        dinv = jnp.where(deg > 0, deg ** -0.5, 0.0)
        norms.append(-dinv[src] * ew * dinv[dst])
    gsrc, gdst = edge_index[0], edge_index[1]
    gdeg = jnp.zeros((N,), jnp.float32).at[gdst].add(1.0) + 1.0
    gdinv = gdeg ** -0.5
    gnorm = gdinv[gsrc] * gdinv[gdst]
    dinv2 = (gdinv * gdinv)[:, None]

    packs = ([_pack_edges(gsrc, gdst, gnorm, GCH)]
             + [_pack_edges(reg_ei[r][0], reg_ei[r][1], norms[r], RCH)
                for r in range(5)])
    spk = jnp.concatenate([a for a, _, _ in packs])
    dpk = jnp.concatenate([b for _, b, _ in packs])
    wpk = jnp.concatenate([c for _, _, c in packs])

    Lr = p['lin5_W'].reshape(5, F_OUT, F_OUT)
    SL = Lr.sum(0)
    C0 = p['cheb_W0'] @ SL
    M = jnp.einsum('ij,rjk->rik', p['cheb_W1'], Lr)
    c1 = p['cheb_b'] @ SL + p['lin5_b']
    Us, Vs, cs = [], [], [c1]
    for g in ['z', 'r', 'h']:
        Wl = p['Wl' + g]
        Us.append(p['W' + g] @ Wl[:F_OUT])
        Vs.append(Wl[F_OUT:])
        cs.append(p['b' + g] @ Wl[:F_OUT] + p['bl' + g])
    U = jnp.stack(Us)
    V = jnp.stack(Vs)
    cvec = jnp.stack(cs)
    probs = jax.nn.softmax(p['att'])[None, :]

    xt = jnp.transpose(x, (2, 0, 1))

    xflat = xt.reshape(PERIODS * N, F_IN)
    zeros_hbm = jnp.zeros((ZR, F_IN), jnp.float32)
    O_a = _spmm_sc(xflat, spk, dpk, wpk, zeros_hbm, 0)
    O_b = _spmm_sc(xflat, spk, dpk, wpk, zeros_hbm, 1)
    O = jnp.concatenate([O_a, O_b]).reshape(PERIODS, 6, N, F_IN)

    out, hid = _dense_stage(xt, O, dinv2, C0, M, U, V, cvec, probs,
                            p['W1'], p['b1'][None, :], p['W2'], p['b2'][None, :])
    return out, hid


# speedup vs baseline: 1.3830x; 1.3830x over previous
import functools
import jax
import jax.numpy as jnp
from jax import lax
from jax.experimental import pallas as pl
from jax.experimental.pallas import tpu as pltpu
from jax.experimental.pallas import tpu_sc as plsc

N = 10000
E = 160000
ER = 32000
F_IN = 128
F_OUT = 256
HIDDEN = 128
OUT_DIM = 12
PERIODS = 12

ROWS = 400
GRID = N // ROWS


def _dense_body(xt_ref, O_ref, dinv2_ref, C0_ref, M_ref, U_ref, V_ref,
                cvec_ref, probs_ref, W1_ref, b1_ref, W2_ref, b2_ref,
                out_ref, hid_ref):
    f32 = jnp.float32
    Hacc = jnp.zeros((ROWS, F_OUT), f32)
    c1 = cvec_ref[0:1, :]
    cz = cvec_ref[1:2, :]
    cr = cvec_ref[2:3, :]
    ch = cvec_ref[3:4, :]
    for t in range(PERIODS):
        xp = xt_ref[t]
        h = jnp.dot(xp, C0_ref[...], preferred_element_type=f32) + c1
        for r in range(5):
            h = h + jnp.dot(O_ref[t, 1 + r], M_ref[r], preferred_element_type=f32)
        h = jnp.where(h >= 0, h, 0.01 * h)
        g = O_ref[t, 0] + dinv2_ref[...] * xp
        Z = jax.nn.sigmoid(jnp.dot(g, U_ref[0], preferred_element_type=f32)
                           + jnp.dot(h, V_ref[0], preferred_element_type=f32) + cz)
        Rg = jax.nn.sigmoid(jnp.dot(g, U_ref[1], preferred_element_type=f32)
                            + jnp.dot(h, V_ref[1], preferred_element_type=f32) + cr)
        Ht = jnp.tanh(jnp.dot(g, U_ref[2], preferred_element_type=f32)
                      + jnp.dot(h * Rg, V_ref[2], preferred_element_type=f32) + ch)
        Hn = Z * h + (1.0 - Z) * Ht
        Hacc = Hacc + probs_ref[0, t] * Hn
    hid_ref[...] = Hacc
    hh = jnp.maximum(Hacc, 0.0)
    hh = jnp.maximum(jnp.dot(hh, W1_ref[...], preferred_element_type=f32)
                     + b1_ref[...], 0.0)
    out_ref[...] = jnp.dot(hh, W2_ref[...], preferred_element_type=f32) + b2_ref[...]


def _dense_stage(xt, O, dinv2, C0, M, U, V, cvec, probs, W1, b1, W2, b2,
                 interpret=False):
    full = lambda *shape: pl.BlockSpec(shape, lambda i: (0,) * len(shape))
    return pl.pallas_call(
        _dense_body,
        grid=(GRID,),
        in_specs=[
            pl.BlockSpec((PERIODS, ROWS, F_IN), lambda i: (0, i, 0)),
            pl.BlockSpec((PERIODS, 6, ROWS, F_IN), lambda i: (0, 0, i, 0)),
            pl.BlockSpec((ROWS, 1), lambda i: (i, 0)),
            full(F_IN, F_OUT),
            full(5, F_IN, F_OUT),
            full(3, F_IN, F_OUT),
            full(3, F_OUT, F_OUT),
            full(4, F_OUT),
            full(1, PERIODS),
            full(F_OUT, HIDDEN),
            full(1, HIDDEN),
            full(HIDDEN, OUT_DIM),
            full(1, OUT_DIM),
        ],
        out_specs=[
            pl.BlockSpec((ROWS, OUT_DIM), lambda i: (i, 0)),
            pl.BlockSpec((ROWS, F_OUT), lambda i: (i, 0)),
        ],
        out_shape=[
            jax.ShapeDtypeStruct((N, OUT_DIM), jnp.float32),
            jax.ShapeDtypeStruct((N, F_OUT), jnp.float32),
        ],
        interpret=interpret,
    )(xt, O, dinv2, C0, M, U, V, cvec, probs, W1, b1, W2, b2)


CH = 128
NS = 16
TROWS = 624
ZR = 104
GCH = 1280
RCH = 256
TOTCH = GCH + 5 * RCH


def _spmm_body(xflat, spk, dpk, wpk, zeros_hbm, o_hbm,
               s0, s1, d0, d1, w0, w1, r0b, r1b, zbuf, acc,
               semE0, semE1, semG0, semG1, semS0, semS1):
    cid = lax.axis_index("c")
    sid = lax.axis_index("s")
    row0 = pl.multiple_of(sid * TROWS, 8)
    pltpu.sync_copy(zeros_hbm, zbuf)

    def scale(rb, wb):
        def grp(g, _):
            wv = wb[pl.ds(16 * g, 16)]
            for l in range(16):
                we = wv[l]
                e = 16 * g + l
                for j in range(F_IN // 16):
                    rb[e, pl.ds(16 * j, 16)] = rb[e, pl.ds(16 * j, 16)] * we
            return 0
        lax.fori_loop(0, CH // 16, grp, 0, unroll=True)

    def prep(sb, toff):
        for j in range(CH // 16):
            sb[pl.ds(16 * j, 16)] = sb[pl.ds(16 * j, 16)] + toff

    def do_pass(t, out_base, cbase, niter):
        for k in range(6):
            pltpu.sync_copy(zbuf, acc.at[pl.ds(row0 + ZR * k, ZR)])
        @pl.when(sid == NS - 1)
        def _():
            pltpu.sync_copy(zbuf.at[pl.ds(0, 16)], acc.at[pl.ds(row0 + 6 * ZR, 16)])
        plsc.subcore_barrier()

        toff = t * N

        def startE(i, sb, db, wb, sem):
            c = pl.multiple_of((cbase + sid + NS * i) * CH, 8)
            pltpu.async_copy(spk.at[pl.ds(c, CH)], sb, sem)
            pltpu.async_copy(dpk.at[pl.ds(c, CH)], db, sem)
            pltpu.async_copy(wpk.at[pl.ds(c, CH)], wb, sem)

        def waitE(sb, db, wb, sem):
            pltpu.make_async_copy(spk.at[pl.ds(0, CH)], sb, sem).wait()
            pltpu.make_async_copy(dpk.at[pl.ds(0, CH)], db, sem).wait()
            pltpu.make_async_copy(wpk.at[pl.ds(0, CH)], wb, sem).wait()

        startG = lambda sb, rb, sem: pltpu.async_copy(xflat.at[sb], rb, sem)

        startE(0, s0, d0, w0, semE0)
        startE(1, s1, d1, w1, semE1)
        waitE(s0, d0, w0, semE0)
        prep(s0, toff)
        startG(s0, r0b, semG0)

        def lbody(i2, _):
            iA = 2 * i2
            pltpu.make_async_copy(xflat.at[s0], r0b, semG0).wait()
            waitE(s1, d1, w1, semE1)
            prep(s1, toff)
            @pl.when(iA > 0)
            def _():
                pltpu.make_async_copy(r1b, acc.at[d1], semS1).wait()
            startG(s1, r1b, semG1)
            scale(r0b, w0)
            pltpu.async_copy(r0b, acc.at[d0], semS0, add=True)
            @pl.when(iA + 2 < niter)
            def _():
                startE(iA + 2, s0, d0, w0, semE0)
            pltpu.make_async_copy(xflat.at[s1], r1b, semG1).wait()
            @pl.when(iA + 2 < niter)
            def _():
                waitE(s0, d0, w0, semE0)
                prep(s0, toff)
                pltpu.make_async_copy(r0b, acc.at[d0], semS0).wait()
                startG(s0, r0b, semG0)
            scale(r1b, w1)
            pltpu.async_copy(r1b, acc.at[d1], semS1, add=True)
            @pl.when(iA + 3 < niter)
            def _():
                startE(iA + 3, s1, d1, w1, semE1)
            return 0
        lax.fori_loop(0, niter // 2, lbody, 0)
        pltpu.make_async_copy(r0b, acc.at[d0], semS0).wait()
        pltpu.make_async_copy(r1b, acc.at[d1], semS1).wait()
        plsc.subcore_barrier()

        ob = pl.multiple_of(out_base + row0, 8)
        @pl.when(sid < NS - 1)
        def _():
            pltpu.sync_copy(acc.at[pl.ds(row0, TROWS)], o_hbm.at[pl.ds(ob, TROWS)])
        @pl.when(sid == NS - 1)
        def _():
            pltpu.sync_copy(acc.at[pl.ds(row0, TROWS + 16)],
                            o_hbm.at[pl.ds(ob, TROWS + 16)])
        plsc.subcore_barrier()

    def per_t(tp, _):
        t = cid * 6 + tp
        do_pass(t, (t * 6) * N, 0, GCH // NS)
        def per_r(r, _):
            do_pass(t, (t * 6 + 1 + r) * N, GCH + RCH * r, RCH // NS)
            return 0
        lax.fori_loop(0, 5, per_r, 0)
        return 0
    lax.fori_loop(0, 6, per_t, 0)


def _spmm_sc(xflat, spk, dpk, wpk, zeros_hbm):
    mesh = plsc.VectorSubcoreMesh(core_axis_name="c", subcore_axis_name="s")
    f = pl.kernel(
        _spmm_body,
        out_type=jax.ShapeDtypeStruct((PERIODS * 6 * N, F_IN), jnp.float32),
        mesh=mesh,
        scratch_types=[
            pltpu.VMEM((CH,), jnp.int32),
            pltpu.VMEM((CH,), jnp.int32),
            pltpu.VMEM((CH,), jnp.int32),
            pltpu.VMEM((CH,), jnp.int32),
            pltpu.VMEM((CH,), jnp.float32),
            pltpu.VMEM((CH,), jnp.float32),
            pltpu.VMEM((CH, F_IN), jnp.float32),
            pltpu.VMEM((CH, F_IN), jnp.float32),
            pltpu.VMEM((ZR, F_IN), jnp.float32),
            pltpu.VMEM_SHARED((N, F_IN), jnp.float32),
            pltpu.SemaphoreType.DMA,
            pltpu.SemaphoreType.DMA,
            pltpu.SemaphoreType.DMA,
            pltpu.SemaphoreType.DMA,
            pltpu.SemaphoreType.DMA,
            pltpu.SemaphoreType.DMA,
        ],
    )
    return f(xflat, spk, dpk, wpk, zeros_hbm)


def _pack_edges(src, dst, w, nch):
    npad = nch * CH - src.shape[0]
    src = jnp.concatenate([src, jnp.zeros((npad,), jnp.int32)])
    dst = jnp.concatenate([dst, jnp.zeros((npad,), jnp.int32)])
    w = jnp.concatenate([w, jnp.zeros((npad,), jnp.float32)])
    return src, dst, w


def kernel(x, edge_index, IAedge_index, KSedge_index, KYedge_index,
           OHedge_index, WIedge_index, IAedge_attr, KSedge_attr, KYedge_attr,
           OHedge_attr, WIedge_attr, params):
    p = params
    reg_ei = (IAedge_index, KSedge_index, KYedge_index, OHedge_index, WIedge_index)
    reg_ew = (IAedge_attr, KSedge_attr, KYedge_attr, OHedge_attr, WIedge_attr)

    norms = []
    for ei, ew in zip(reg_ei, reg_ew):
        src, dst = ei[0], ei[1]
        deg = jnp.zeros((N,), jnp.float32).at[src].add(ew)
        dinv = jnp.where(deg > 0, deg ** -0.5, 0.0)
        norms.append(-dinv[src] * ew * dinv[dst])
    gsrc, gdst = edge_index[0], edge_index[1]
    gdeg = jnp.zeros((N,), jnp.float32).at[gdst].add(1.0) + 1.0
    gdinv = gdeg ** -0.5
    gnorm = gdinv[gsrc] * gdinv[gdst]
    dinv2 = (gdinv * gdinv)[:, None]

    packs = ([_pack_edges(gsrc, gdst, gnorm, GCH)]
             + [_pack_edges(reg_ei[r][0], reg_ei[r][1], norms[r], RCH)
                for r in range(5)])
    spk = jnp.concatenate([a for a, _, _ in packs])
    dpk = jnp.concatenate([b for _, b, _ in packs])
    wpk = jnp.concatenate([c for _, _, c in packs])

    Lr = p['lin5_W'].reshape(5, F_OUT, F_OUT)
    SL = Lr.sum(0)
    C0 = p['cheb_W0'] @ SL
    M = jnp.einsum('ij,rjk->rik', p['cheb_W1'], Lr)
    c1 = p['cheb_b'] @ SL + p['lin5_b']
    Us, Vs, cs = [], [], [c1]
    for g in ['z', 'r', 'h']:
        Wl = p['Wl' + g]
        Us.append(p['W' + g] @ Wl[:F_OUT])
        Vs.append(Wl[F_OUT:])
        cs.append(p['b' + g] @ Wl[:F_OUT] + p['bl' + g])
    U = jnp.stack(Us)
    V = jnp.stack(Vs)
    cvec = jnp.stack(cs)
    probs = jax.nn.softmax(p['att'])[None, :]

    xt = jnp.transpose(x, (2, 0, 1))

    xflat = xt.reshape(PERIODS * N, F_IN)
    zeros_hbm = jnp.zeros((ZR, F_IN), jnp.float32)
    O = _spmm_sc(xflat, spk, dpk, wpk, zeros_hbm)
    O = O.reshape(PERIODS, 6, N, F_IN)

    out, hid = _dense_stage(xt, O, dinv2, C0, M, U, V, cvec, probs,
                            p['W1'], p['b1'][None, :], p['W2'], p['b2'][None, :])
    return out, hid
